# Initial kernel scaffold; baseline (speedup 1.0000x reference)
import jax
import jax.numpy as jnp
from jax.experimental import pallas as pl


def kernel(x, edge_index, node_type, edge_type, params):
    raise NotImplementedError("write your pallas kernel here")


# speedup vs baseline: 1.0002x; 1.0002x over previous
import functools

import jax
import jax.numpy as jnp
from jax.experimental import pallas as pl

H = 8
C = 16


def _leaky(x, s=0.01):
    return jnp.where(x >= 0, x, s * x)


def _dot_t(a, w):
    return jax.lax.dot_general(a, w, (((1,), (1,)), ((), ())),
                               preferred_element_type=jnp.float32)


def _mlp_body(x_ref, g1_ref, b1_ref, w1_ref, c1_ref, g2_ref, b2_ref, w2_ref,
              c2_ref, o_ref):
    x = x_ref[...]
    m = jnp.mean(x, axis=0, keepdims=True)
    v = jnp.mean((x - m) ** 2, axis=0, keepdims=True)
    x = (x - m) * jax.lax.rsqrt(v + 1e-5) * g1_ref[...] + b1_ref[...]
    h = _leaky(_dot_t(x, w1_ref[...]) + c1_ref[...])
    m2 = jnp.mean(h, axis=0, keepdims=True)
    v2 = jnp.mean((h - m2) ** 2, axis=0, keepdims=True)
    h = (h - m2) * jax.lax.rsqrt(v2 + 1e-5) * g2_ref[...] + b2_ref[...]
    o_ref[...] = _leaky(_dot_t(h, w2_ref[...]) + c2_ref[...])


def _mlp_pallas(x, p):
    n = x.shape[0]
    outd = p['W2'].shape[0]
    return pl.pallas_call(
        _mlp_body,
        out_shape=jax.ShapeDtypeStruct((n, outd), jnp.float32),
    )(x, p['g1'][None], p['b1'][None], p['W1'], p['c1'][None],
      p['g2'][None], p['b2'][None], p['W2'], p['c2'][None])


def _gatv2_jax(x, edge_index, edge_attr, p):
    n = x.shape[0]
    src, dst = edge_index[0], edge_index[1]
    sums = jax.ops.segment_sum(edge_attr, dst, num_segments=n)
    cnt = jax.ops.segment_sum(jnp.ones((edge_attr.shape[0],), dtype=x.dtype),
                              dst, num_segments=n)
    loop_attr = sums / jnp.maximum(cnt, 1.0)[:, None]
    ar = jnp.arange(n, dtype=src.dtype)
    src = jnp.concatenate([src, ar])
    dst = jnp.concatenate([dst, ar])
    ea = jnp.concatenate([edge_attr, loop_attr], 0)
    xl = (x @ p['Wl'].T + p['bl']).reshape(n, H, C)
    xr = (x @ p['Wr'].T + p['br']).reshape(n, H, C)
    ef = (ea @ p['We'].T).reshape(-1, H, C)
    e = _leaky(xl[src] + xr[dst] + ef, 0.2)
    logit = (e * p['att'][None]).sum(-1)
    m = jax.ops.segment_max(logit, dst, num_segments=n)
    m = jnp.where(jnp.isfinite(m), m, 0.0)
    a = jnp.exp(logit - m[dst])
    den = jax.ops.segment_sum(a, dst, num_segments=n)
    a = a / (den[dst] + 1e-16)
    out = jax.ops.segment_sum(xl[src] * a[..., None], dst, num_segments=n)
    return out.reshape(n, H * C) + p['bias']


def kernel(x, edge_index, node_type, edge_type, params):
    h = jnp.concatenate([x, params['node_emb'][node_type]], 1)
    ee = params['edge_emb'][edge_type]
    h = _mlp_pallas(h, params['in_mlp'])
    for blk in params['blocks']:
        h = h + _mlp_pallas(_gatv2_jax(h, edge_index, ee, blk['conv']),
                            blk['mlp'])
    h = h[:4278]
    return _mlp_pallas(h, params['out_mlp'])


# speedup vs baseline: 17.7948x; 17.7916x over previous
import functools

import jax
import jax.numpy as jnp
from jax import lax
from jax.experimental import pallas as pl
from jax.experimental.pallas import tpu as pltpu, tpu_sc as plsc

N = 10000
E = 320000
D = 128
H = 8
C = 16
NC = 2
NS = 16
NW = NC * NS
EPT = E // NW
K = 80
NCHUNK = EPT // K
NP = 10112
ROWS_PER_SUB = NP // NS


def _leaky(x, s):
    return jnp.where(x >= 0, x, s * x)


def _dot_t(a, w):
    return jax.lax.dot_general(a, w, (((1,), (1,)), ((), ())),
                               preferred_element_type=jnp.float32)


def _dot(a, w):
    return jax.lax.dot_general(a, w, (((1,), (0,)), ((), ())),
                               preferred_element_type=jnp.float32,
                               precision=jax.lax.Precision.HIGHEST)


def _bn_lin_body(x_ref, m, v, g, b, w, c, o_ref):
    xn = (x_ref[...] - m[...]) / jnp.sqrt(v[...] + 1e-5) * g[...] + b[...]
    o_ref[...] = _leaky(_dot_t(xn, w[...]) + c[...], 0.01)


def _bn_lin_res_body(x_ref, m, v, g, b, w, c, h_ref, o_ref):
    xn = (x_ref[...] - m[...]) / jnp.sqrt(v[...] + 1e-5) * g[...] + b[...]
    o_ref[...] = h_ref[...] + _leaky(_dot_t(xn, w[...]) + c[...], 0.01)


def _bn_lin(x, m, v, g, b, w, c, res=None):
    n = x.shape[0]
    outd = w.shape[0]
    if res is None:
        return pl.pallas_call(
            _bn_lin_body,
            out_shape=jax.ShapeDtypeStruct((n, outd), jnp.float32),
        )(x, m, v, g[None], b[None], w, c[None])
    return pl.pallas_call(
        _bn_lin_res_body,
        out_shape=jax.ShapeDtypeStruct((n, outd), jnp.float32),
    )(x, m, v, g[None], b[None], w, c[None], res)


def _stats(x):
    m = x.mean(0)
    v = ((x - m) ** 2).mean(0)
    return m[None], v[None]


def _mlp_pallas(x, p, res=None):
    m1, v1 = _stats(x)
    hid = _bn_lin(x, m1, v1, p['g1'], p['b1'], p['W1'], p['c1'])
    m2, v2 = _stats(hid)
    return _bn_lin(hid, m2, v2, p['g2'], p['b2'], p['W2'], p['c2'], res=res)


def _mlp_res_pallas(x, h, p):
    return _mlp_pallas(x, p, res=h)


def _pre_body(h_ref, wl, bl, wr, br, emb, we, xl_ref, xr_ref, ef8_ref):
    hh = h_ref[...]
    xl_ref[...] = _dot_t(hh, wl[...]) + bl[...]
    xr_ref[...] = _dot_t(hh, wr[...]) + br[...]
    ef8_ref[...] = _dot_t(emb[...], we[...])


def _tc_pre(h, cp, edge_emb):
    return pl.pallas_call(
        _pre_body,
        out_shape=(jax.ShapeDtypeStruct((N, D), jnp.float32),
                   jax.ShapeDtypeStruct((N, D), jnp.float32),
                   jax.ShapeDtypeStruct((8, D), jnp.float32)),
    )(h, cp['Wl'], cp['bl'][None], cp['Wr'], cp['br'][None], edge_emb,
      cp['We'])


def _conv_body(acc_ref, accA_ref, xl_ref, xr_ref, we, attf, bias, o_ref):
    accs = acc_ref[0] + acc_ref[1]
    num = accs[:, :D]
    den8 = accs[:, D:D + H]
    sA = accA_ref[0] + accA_ref[1]
    sums = sA[:, :C]
    cnt = sA[:, C:C + 1]
    loop_attr = sums / jnp.maximum(cnt, 1.0)
    ef_loop = _dot_t(loop_attr, we[...])
    xl = xl_ref[...]
    e = _leaky(xl + xr_ref[...] + ef_loop, 0.2)
    lw = e * attf[...]
    ii = lax.broadcasted_iota(jnp.int32, (D, D), 0) // C
    jj = lax.broadcasted_iota(jnp.int32, (D, D), 1) // C
    M = (ii == jj).astype(jnp.float32)
    pw = jnp.exp(_dot(lw, M))
    i8 = lax.broadcasted_iota(jnp.int32, (H, D), 0)
    j8 = lax.broadcasted_iota(jnp.int32, (H, D), 1) // C
    R = (i8 == j8).astype(jnp.float32)
    denw = _dot(den8, R) + pw
    o_ref[...] = (num + xl * pw) / denw + bias[...]


_CONV_B = 2000


def _tc_conv(acc, accA, xl, xr, cp):
    attf = cp['att'].reshape(1, H * C)
    nb = N // _CONV_B
    return pl.pallas_call(
        _conv_body,
        grid=(nb,),
        in_specs=[
            pl.BlockSpec((2, _CONV_B, D + C), lambda i: (0, i, 0)),
            pl.BlockSpec((2, _CONV_B, 2 * C), lambda i: (0, i, 0)),
            pl.BlockSpec((_CONV_B, D), lambda i: (i, 0)),
            pl.BlockSpec((_CONV_B, D), lambda i: (i, 0)),
            pl.BlockSpec((D, C), lambda i: (0, 0)),
            pl.BlockSpec((1, D), lambda i: (0, 0)),
            pl.BlockSpec((1, D), lambda i: (0, 0)),
        ],
        out_specs=pl.BlockSpec((_CONV_B, D), lambda i: (i, 0)),
        out_shape=jax.ShapeDtypeStruct((N, D), jnp.float32),
    )(acc, accA, xl, xr, cp['We'], attf, cp['bias'][None])


@functools.cache
def _mesh():
    return plsc.VectorSubcoreMesh(core_axis_name="c", subcore_axis_name="s")


def _sc_main_body(xl_hbm, xr_hbm, src_hbm, dst_hbm, et_hbm, ef8_hbm,
                  att_hbm, zero_hbm, out_hbm,
                  src_v, dst_v, et_v, xl_rows, xr_rows, contrib, ef8_v, att_v,
                  acc, gs0):
    cid = lax.axis_index("c")
    sid = lax.axis_index("s")
    wid = cid * NS + sid

    pltpu.sync_copy(zero_hbm.at[pl.ds(sid * ROWS_PER_SUB, ROWS_PER_SUB)],
                    acc.at[pl.ds(sid * ROWS_PER_SUB, ROWS_PER_SUB)])
    pltpu.sync_copy(ef8_hbm, ef8_v)
    pltpu.sync_copy(att_hbm, att_v)

    zvec = jnp.zeros((C,), jnp.float32)
    for g in range(K // C):
        eoff0 = g * C + lax.iota(jnp.int32, C)
        for col in range(D + H, D + C):
            plsc.store_scatter(
                contrib, [eoff0, jnp.full((C,), col, jnp.int32)], zvec)

    plsc.subcore_barrier()

    def chunk_body(j, carry):
        base = wid * EPT + j * K
        pltpu.sync_copy(src_hbm.at[pl.ds(base, K)], src_v)
        pltpu.sync_copy(dst_hbm.at[pl.ds(base, K)], dst_v.at[0])
        pltpu.sync_copy(et_hbm.at[pl.ds(base, K)], et_v)
        pltpu.async_copy(xl_hbm.at[src_v], xl_rows, gs0)
        pltpu.async_copy(xr_hbm.at[dst_v.at[0]], xr_rows, gs0)
        pltpu.make_async_copy(xl_hbm.at[src_v], xl_rows, gs0).wait()
        pltpu.make_async_copy(xr_hbm.at[dst_v.at[0]], xr_rows, gs0).wait()

        def group_body(g, carry2):
            eoff = g * C + lax.iota(jnp.int32, C)
            et_g = plsc.load_gather(et_v, [eoff])
            for h in range(H):
                arow = att_v[pl.ds(h * C, C)]
                logit = jnp.zeros((C,), jnp.float32)
                avs = []
                for c in range(C):
                    cvec = jnp.full((C,), h * C + c, jnp.int32)
                    a = plsc.load_gather(xl_rows, [eoff, cvec])
                    avs.append(a)
                    b = plsc.load_gather(xr_rows, [eoff, cvec])
                    f = plsc.load_gather(ef8_v, [et_g, cvec])
                    s = a + b + f
                    s = jnp.where(s >= 0, s, 0.2 * s)
                    logit = logit + s * arow[c]
                p = jnp.exp(logit)
                plsc.store_scatter(
                    contrib, [eoff, jnp.full((C,), D + h, jnp.int32)], p)
                for c in range(C):
                    cvec = jnp.full((C,), h * C + c, jnp.int32)
                    plsc.store_scatter(contrib, [eoff, cvec], avs[c] * p)
            return carry2

        lax.fori_loop(0, K // C, group_body, 0, unroll=False)
        pltpu.sync_copy(contrib, acc.at[dst_v.at[0]], add=True)
        return carry

    lax.fori_loop(0, NCHUNK, chunk_body, 0, unroll=False)

    plsc.subcore_barrier()
    pltpu.sync_copy(acc.at[pl.ds(sid * ROWS_PER_SUB, ROWS_PER_SUB)],
                    out_hbm.at[cid, pl.ds(sid * ROWS_PER_SUB, ROWS_PER_SUB)])


_SC_PARAMS = pltpu.CompilerParams(use_tc_tiling_on_sc=False,
                                  needs_layout_passes=False)


@functools.cache
def _sc_main_kernel():
    return pl.kernel(
    _sc_main_body,
    out_type=jax.ShapeDtypeStruct((NC, NP, D + C), jnp.float32),
    mesh=_mesh(),
    compiler_params=_SC_PARAMS,
    scratch_types=[
        pltpu.VMEM((K,), jnp.int32),
        pltpu.VMEM((1, K), jnp.int32),
        pltpu.VMEM((K,), jnp.int32),
        pltpu.VMEM((K, D), jnp.float32),
        pltpu.VMEM((K, D), jnp.float32),
        pltpu.VMEM((K, D + C), jnp.float32),
        pltpu.VMEM((8, D), jnp.float32),
        pltpu.VMEM((H * C,), jnp.float32),
        pltpu.VMEM_SHARED((NP, D + C), jnp.float32),
        pltpu.SemaphoreType.DMA,
    ],
    )


def _sc_loopattr_body(dst_hbm, et_hbm, emb_hbm, zero_hbm, out_hbm,
                      dst_v, et_v, contrib, emb_v, acc):
    cid = lax.axis_index("c")
    sid = lax.axis_index("s")
    wid = cid * NS + sid
    rps = ROWS_PER_SUB
    pltpu.sync_copy(zero_hbm.at[pl.ds(sid * rps, rps)],
                    acc.at[pl.ds(sid * rps, rps)])
    pltpu.sync_copy(emb_hbm, emb_v)

    zvec = jnp.zeros((C,), jnp.float32)
    ones = jnp.ones((C,), jnp.float32)
    for g in range(K // C):
        eoff0 = g * C + lax.iota(jnp.int32, C)
        for col in range(C + 1, 2 * C):
            plsc.store_scatter(contrib,
                               [eoff0, jnp.full((C,), col, jnp.int32)], zvec)

    plsc.subcore_barrier()

    def chunk_body(j, carry):
        base = wid * EPT + j * K
        pltpu.sync_copy(dst_hbm.at[pl.ds(base, K)], dst_v)
        pltpu.sync_copy(et_hbm.at[pl.ds(base, K)], et_v)

        def group_body(g, carry2):
            eoff = g * C + lax.iota(jnp.int32, C)
            et_g = plsc.load_gather(et_v, [eoff])
            for c in range(C):
                cvec = jnp.full((C,), c, jnp.int32)
                v = plsc.load_gather(emb_v, [et_g, cvec])
                plsc.store_scatter(contrib, [eoff, cvec], v)
            plsc.store_scatter(contrib, [eoff, jnp.full((C,), C, jnp.int32)],
                               ones)
            return carry2

        lax.fori_loop(0, K // C, group_body, 0, unroll=False)
        pltpu.sync_copy(contrib, acc.at[dst_v], add=True)
        return carry

    lax.fori_loop(0, NCHUNK, chunk_body, 0, unroll=False)

    plsc.subcore_barrier()
    pltpu.sync_copy(acc.at[pl.ds(sid * rps, rps)],
                    out_hbm.at[cid, pl.ds(sid * rps, rps)])


@functools.cache
def _sc_loopattr_kernel():
    return pl.kernel(
    _sc_loopattr_body,
    out_type=jax.ShapeDtypeStruct((NC, NP, 2 * C), jnp.float32),
    mesh=_mesh(),
    compiler_params=_SC_PARAMS,
    scratch_types=[
        pltpu.VMEM((K,), jnp.int32),
        pltpu.VMEM((K,), jnp.int32),
        pltpu.VMEM((K, 2 * C), jnp.float32),
        pltpu.VMEM((8, C), jnp.float32),
        pltpu.VMEM_SHARED((NP, 2 * C), jnp.float32),
    ],
    )


def kernel(x, edge_index, node_type, edge_type, params):
    src = edge_index[0]
    dst = edge_index[1]
    et = edge_type.astype(jnp.int32)

    h = jnp.concatenate([x, params['node_emb'][node_type]], 1)
    h = _mlp_pallas(h, params['in_mlp'])

    zero144 = jnp.zeros((NP, D + C), jnp.float32)
    zero32 = jnp.zeros((NP, 2 * C), jnp.float32)
    accA = _sc_loopattr_kernel()(dst, et, params['edge_emb'], zero32)[:, :N]

    for blk in params['blocks']:
        cp = blk['conv']
        xl, xr, ef8 = _tc_pre(h, cp, params['edge_emb'])
        acc = _sc_main_kernel()(xl, xr, src, dst, et, ef8,
                                cp['att'].reshape(H * C), zero144)[:, :N]
        conv = _tc_conv(acc, accA, xl, xr, cp)
        h = _mlp_res_pallas(conv, h, blk['mlp'])

    return _mlp_pallas(h[:4278], params['out_mlp'])
